# baseline (speedup 1.0000x reference)
import functools

import jax
import jax.numpy as jnp
import numpy as np
from jax import lax
from jax.experimental import pallas as pl
from jax.experimental.pallas import tpu as pltpu
from jax.experimental.pallas import tpu_sc as plsc

NV = 5000
NCON = 5000
NN = 10000
E = 320000
H = 128

RB = 1000
EB = 2560
ER = 20
NEB = E // EB

SC_NC = 2
SC_NS = 16
EC = E // SC_NC
EPW = EC // SC_NS
K = 80
NCHUNK = EPW // K
RPS = 624

_f32 = jnp.float32
_bf16 = jnp.bfloat16

_P = np.empty((H,), dtype=np.int32)
for _cb in range(H // 32):
    for _i in range(16):
        for _h in range(2):
            _P[32 * _cb + 2 * _i + _h] = 32 * _cb + 16 * _h + _i
_PINV = np.argsort(_P).astype(np.int32)


def _full(shape):
    return pl.BlockSpec(shape, lambda i: tuple(0 for _ in shape))


def _enc_body(x_ref, w1_ref, b1_ref, w2_ref, b2_ref, o_ref):
    h = jnp.maximum(
        jnp.dot(x_ref[...], w1_ref[...], preferred_element_type=_f32)
        + b1_ref[...], 0.0)
    o_ref[...] = (
        jnp.dot(h, w2_ref[...], preferred_element_type=_f32) + b2_ref[...])


def _enc(xp, w1p, b1, w2, b2):
    n = xp.shape[0]
    return pl.pallas_call(
        _enc_body,
        grid=(n // RB,),
        in_specs=[
            pl.BlockSpec((RB, H), lambda i: (i, 0)),
            _full((H, H)), _full((1, H)), _full((H, H)), _full((1, H)),
        ],
        out_specs=pl.BlockSpec((RB, H), lambda i: (i, 0)),
        out_shape=jax.ShapeDtypeStruct((n, H), _f32),
    )(xp, w1p, b1, w2, b2)


def _eemb_body(ea_ref, w1_ref, b1_ref, w2_ref, b2_ref, g_ref, bt_ref,
               h2_ref, ab_ref, acc_ref):
    i = pl.program_id(0)

    @pl.when(i == 0)
    def _():
        acc_ref[...] = jnp.zeros((8, H), _f32)

    a = ea_ref[0]
    abc = jnp.broadcast_to(a[:, :, None], (ER, 128, H)).reshape(EB, H)
    h1 = jnp.maximum(abc * w1_ref[...] + b1_ref[...], 0.0)
    h2 = jnp.maximum(
        jnp.dot(h1, w2_ref[...], preferred_element_type=_f32)
        + b2_ref[...], 0.0)
    h2_ref[...] = h2
    acc_ref[0:1] = acc_ref[0:1] + jnp.sum(h2, axis=0, keepdims=True)
    acc_ref[1:2] = acc_ref[1:2] + jnp.sum(h2 * h2, axis=0, keepdims=True)

    @pl.when(i == NEB - 1)
    def _():
        mu = acc_ref[0:1] / E
        var = acc_ref[1:2] / E - mu * mu
        aa = g_ref[...] * lax.rsqrt(var + 1e-5)
        bb = bt_ref[...] - mu * aa
        ab_ref[...] = jnp.concatenate(
            [aa, bb, jnp.zeros((6, H), _f32)], axis=0)


def _eemb(ea3, w1, b1, w2, b2, g, bt):
    return pl.pallas_call(
        _eemb_body,
        grid=(NEB,),
        in_specs=[
            pl.BlockSpec((1, ER, 128), lambda i: (i, 0, 0)),
            _full((1, H)), _full((1, H)), _full((H, H)), _full((1, H)),
            _full((1, H)), _full((1, H)),
        ],
        out_specs=[
            pl.BlockSpec((EB, H), lambda i: (i, 0)),
            _full((8, H)),
        ],
        out_shape=[
            jax.ShapeDtypeStruct((E, H), _f32),
            jax.ShapeDtypeStruct((8, H), _f32),
        ],
        scratch_shapes=[pltpu.VMEM((8, H), _f32)],
    )(ea3, w1, b1, w2, b2, g, bt)


def _upd_body(x_ref, agg_ref, eps_ref, w1_ref, b1_ref, w2_ref, b2_ref,
              g_ref, bt_ref, h2_ref, ab_ref, acc_ref):
    i = pl.program_id(0)

    @pl.when(i == 0)
    def _():
        acc_ref[...] = jnp.zeros((8, H), _f32)

    pre = x_ref[...] * (1.0 + eps_ref[...]) + agg_ref[0] + agg_ref[1]
    h1 = jnp.maximum(
        jnp.dot(pre, w1_ref[...], preferred_element_type=_f32)
        + b1_ref[...], 0.0)
    h2 = jnp.maximum(
        jnp.dot(h1, w2_ref[...], preferred_element_type=_f32)
        + b2_ref[...], 0.0)
    h2_ref[...] = h2
    acc_ref[0:1] = acc_ref[0:1] + jnp.sum(h2, axis=0, keepdims=True)
    acc_ref[1:2] = acc_ref[1:2] + jnp.sum(h2 * h2, axis=0, keepdims=True)

    @pl.when(i == NN // RB - 1)
    def _():
        mu = acc_ref[0:1] / NN
        var = acc_ref[1:2] / NN - mu * mu
        aa = g_ref[...] * lax.rsqrt(var + 1e-5)
        bb = bt_ref[...] - mu * aa
        ab_ref[...] = jnp.concatenate(
            [aa, bb, jnp.zeros((6, H), _f32)], axis=0)


def _upd(x, aggp, epsb, w1, b1, w2, b2, g, bt):
    return pl.pallas_call(
        _upd_body,
        grid=(NN // RB,),
        in_specs=[
            pl.BlockSpec((RB, H), lambda i: (i, 0)),
            pl.BlockSpec((2, RB, H), lambda i: (0, i, 0)),
            _full((1, H)),
            _full((H, H)), _full((1, H)), _full((H, H)), _full((1, H)),
            _full((1, H)), _full((1, H)),
        ],
        out_specs=[
            pl.BlockSpec((RB, H), lambda i: (i, 0)),
            _full((8, H)),
        ],
        out_shape=[
            jax.ShapeDtypeStruct((NN, H), _f32),
            jax.ShapeDtypeStruct((8, H), _f32),
        ],
        scratch_shapes=[pltpu.VMEM((8, H), _f32)],
    )(x, aggp, epsb, w1, b1, w2, b2, g, bt)


def _apply_body(h_ref, ab_ref, o_ref):
    o_ref[...] = jnp.maximum(h_ref[...] * ab_ref[0:1] + ab_ref[1:2], 0.0)


def _apply(h, ab):
    return pl.pallas_call(
        _apply_body,
        grid=(NN // RB,),
        in_specs=[
            pl.BlockSpec((RB, H), lambda i: (i, 0)),
            _full((8, H)),
        ],
        out_specs=pl.BlockSpec((RB, H), lambda i: (i, 0)),
        out_shape=jax.ShapeDtypeStruct((NN, H), _f32),
    )(h, ab)


def _head_body(x0_ref, x1_ref, x2_ref, x3_ref, x4_ref, w1_ref, b1_ref,
               w2_ref, b2_ref, w3_ref, b3_ref, w4_ref, b4_ref, o_ref):
    w = w1_ref[...]
    h = (jnp.dot(x0_ref[...], w[0:H], preferred_element_type=_f32)
         + jnp.dot(x1_ref[...], w[H:2 * H], preferred_element_type=_f32)
         + jnp.dot(x2_ref[...], w[2 * H:3 * H], preferred_element_type=_f32)
         + jnp.dot(x3_ref[...], w[3 * H:4 * H], preferred_element_type=_f32)
         + jnp.dot(x4_ref[...], w[4 * H:5 * H], preferred_element_type=_f32)
         + b1_ref[...])
    h = jnp.maximum(h, 0.0)
    h = jnp.maximum(
        jnp.dot(h, w2_ref[...], preferred_element_type=_f32)
        + b2_ref[...], 0.0)
    h = jnp.maximum(
        jnp.dot(h, w3_ref[...], preferred_element_type=_f32)
        + b3_ref[...], 0.0)
    of = jnp.dot(h, w4_ref[...], preferred_element_type=_f32)
    o0 = of[:, 0:1] + b4_ref[:, 0:1]
    o1 = of[:, 1:2] + b4_ref[:, 1:2]
    m = jnp.maximum(o0, o1)
    ls = m + jnp.log(jnp.exp(o0 - m) + jnp.exp(o1 - m))
    o_ref[...] = jnp.concatenate(
        [o0 - ls, o1 - ls, jnp.zeros((RB, H - 2), _f32)], axis=1)


def _head(x0, x1, x2, x3, x4, w1, b1, w2, b2, w3, b3, w4p, b4p):
    blk = pl.BlockSpec((RB, H), lambda i: (i, 0))
    return pl.pallas_call(
        _head_body,
        grid=(NV // RB,),
        in_specs=[
            blk, blk, blk, blk, blk,
            _full((5 * H, H)), _full((1, H)),
            _full((H, H)), _full((1, H)),
            _full((H, H)), _full((1, H)),
            _full((H, H)), _full((1, H)),
        ],
        out_specs=pl.BlockSpec((RB, H), lambda i: (i, 0)),
        out_shape=jax.ShapeDtypeStruct((NV, H), _f32),
    )(x0, x1, x2, x3, x4, w1, b1, w2, b2, w3, b3, w4p, b4p)


def _mp_body(x_hbm, h2e_hbm, src_hbm, dst_hbm, ab_hbm, out_hbm,
             sidx_a, sidx_b, didx_v, gx_a, he_a, gx_b, he_b,
             ab_v, agg_sh, s_si_a, s_si_b, s_di, s_g_a, s_g_b, s_h_a, s_h_b):
    cid = lax.axis_index("c")
    sid = lax.axis_index("s")

    def _zrow(r, carry):
        for c in range(H // 16):
            gx_a[r, pl.ds(c * 16, 16)] = jnp.zeros((16,), _f32)
        return carry
    lax.fori_loop(0, K, _zrow, 0)
    start = sid * RPS
    for j in range(RPS // K):
        pltpu.sync_copy(gx_a, agg_sh.at[pl.ds(start + j * K, K)])
    pltpu.sync_copy(gx_a.at[pl.ds(0, RPS - (RPS // K) * K)],
                    agg_sh.at[pl.ds(start + (RPS // K) * K,
                                    RPS - (RPS // K) * K)])

    @pl.when(sid == SC_NS - 1)
    def _():
        pltpu.sync_copy(gx_a.at[pl.ds(0, NN - SC_NS * RPS)],
                        agg_sh.at[pl.ds(SC_NS * RPS, NN - SC_NS * RPS)])

    pltpu.sync_copy(ab_hbm, ab_v)
    avs = [ab_v[0, pl.ds(c * 16, 16)] for c in range(H // 16)]
    bvs = [ab_v[1, pl.ds(c * 16, 16)] for c in range(H // 16)]

    plsc.subcore_barrier()

    base = cid * EC + sid * EPW

    def issue_si(i, si, sem):
        pltpu.async_copy(src_hbm.at[pl.ds(base + i * K, K)], si, sem)

    def wait_si(si, sem):
        pltpu.make_async_copy(src_hbm.at[pl.ds(0, K)], si, sem).wait()

    def issue_di(i):
        pltpu.async_copy(dst_hbm.at[pl.ds(base + i * K, K)], didx_v, s_di)

    def issue_g(i, si, gx, he, sg, sh):
        pltpu.async_copy(x_hbm.at[si], gx, sg)
        pltpu.async_copy(h2e_hbm.at[pl.ds((base + i * K) * H, K * H)], he, sh)

    def msg_loop(gx, he):
        def _row(r, c2):
            for c in range(H // 16):
                g = gx[r, pl.ds(c * 16, 16)]
                hh = he[pl.ds(r * H + c * 16, 16)]
                gx[r, pl.ds(c * 16, 16)] = jnp.maximum(
                    g + (hh * avs[c] + bvs[c]), 0.0)
            return c2
        lax.fori_loop(0, K, _row, 0, unroll=4)

    def compute_scatter(gx, he, sg, sh):
        pltpu.make_async_copy(x_hbm.at[pl.ds(0, K)], gx, sg).wait()
        pltpu.make_async_copy(h2e_hbm.at[pl.ds(0, K * H)], he, sh).wait()
        msg_loop(gx, he)
        pltpu.make_async_copy(dst_hbm.at[pl.ds(0, K)], didx_v, s_di).wait()
        pltpu.sync_copy(gx, agg_sh.at[didx_v], add=True)

    issue_si(0, sidx_a, s_si_a)
    issue_si(1, sidx_b, s_si_b)
    issue_di(0)
    wait_si(sidx_a, s_si_a)
    issue_g(0, sidx_a, gx_a, he_a, s_g_a, s_h_a)

    def _pair(j, carry):
        i0 = 2 * j
        i1 = i0 + 1
        wait_si(sidx_b, s_si_b)
        issue_g(i1, sidx_b, gx_b, he_b, s_g_b, s_h_b)
        pltpu.make_async_copy(x_hbm.at[pl.ds(0, K)], gx_a, s_g_a).wait()
        issue_si(i0 + 2, sidx_a, s_si_a)
        pltpu.make_async_copy(h2e_hbm.at[pl.ds(0, K * H)], he_a, s_h_a).wait()
        msg_loop(gx_a, he_a)
        pltpu.make_async_copy(dst_hbm.at[pl.ds(0, K)], didx_v, s_di).wait()
        pltpu.sync_copy(gx_a, agg_sh.at[didx_v], add=True)
        issue_di(i1)
        wait_si(sidx_a, s_si_a)
        issue_g(i0 + 2, sidx_a, gx_a, he_a, s_g_a, s_h_a)
        pltpu.make_async_copy(x_hbm.at[pl.ds(0, K)], gx_b, s_g_b).wait()

        @pl.when(i1 + 2 < NCHUNK)
        def _():
            issue_si(i1 + 2, sidx_b, s_si_b)
        pltpu.make_async_copy(h2e_hbm.at[pl.ds(0, K * H)], he_b, s_h_b).wait()
        msg_loop(gx_b, he_b)
        pltpu.make_async_copy(dst_hbm.at[pl.ds(0, K)], didx_v, s_di).wait()
        pltpu.sync_copy(gx_b, agg_sh.at[didx_v], add=True)
        issue_di(i0 + 2)
        return carry
    lax.fori_loop(0, (NCHUNK - 1) // 2, _pair, 0)

    compute_scatter(gx_a, he_a, s_g_a, s_h_a)

    plsc.subcore_barrier()

    nfull = RPS // K
    for j in range(nfull):
        r0 = start + j * K
        pltpu.sync_copy(agg_sh.at[pl.ds(r0, K)], gx_a)
        pltpu.sync_copy(gx_a, out_hbm.at[cid, pl.ds(r0, K)])
    rpart = RPS - nfull * K
    r0 = start + nfull * K
    pltpu.sync_copy(agg_sh.at[pl.ds(r0, rpart)], gx_a.at[pl.ds(0, rpart)])
    pltpu.sync_copy(gx_a.at[pl.ds(0, rpart)], out_hbm.at[cid, pl.ds(r0, rpart)])

    @pl.when(sid == SC_NS - 1)
    def _():
        rem = NN - SC_NS * RPS
        pltpu.sync_copy(agg_sh.at[pl.ds(SC_NS * RPS, rem)],
                        gx_b.at[pl.ds(0, rem)])
        pltpu.sync_copy(gx_b.at[pl.ds(0, rem)],
                        out_hbm.at[cid, pl.ds(SC_NS * RPS, rem)])


@functools.lru_cache(maxsize=1)
def _mp_build():
    mesh = plsc.VectorSubcoreMesh(
        core_axis_name="c", subcore_axis_name="s",
        num_cores=SC_NC, num_subcores=SC_NS)
    return pl.kernel(
        _mp_body,
        out_type=jax.ShapeDtypeStruct((SC_NC, NN, H), _f32),
        mesh=mesh,
        scratch_types=[
            pltpu.VMEM((K,), jnp.int32),
            pltpu.VMEM((K,), jnp.int32),
            pltpu.VMEM((K,), jnp.int32),
            pltpu.VMEM((K, H), _f32),
            pltpu.VMEM((K * H,), _f32),
            pltpu.VMEM((K, H), _f32),
            pltpu.VMEM((K * H,), _f32),
            pltpu.VMEM((8, H), _f32),
            pltpu.VMEM_SHARED((NN, H), _f32),
            pltpu.SemaphoreType.DMA,
            pltpu.SemaphoreType.DMA,
            pltpu.SemaphoreType.DMA,
            pltpu.SemaphoreType.DMA,
            pltpu.SemaphoreType.DMA,
            pltpu.SemaphoreType.DMA,
            pltpu.SemaphoreType.DMA,
        ],
    )


def _mp_call(x, h2e, src, dst, abe):
    return _mp_build()(x, h2e, src, dst, abe)


def kernel(var_node_features, con_node_features, edge_features, edge_index,
           assoc_var, assoc_con, vW1, vb1, vW2, vb2, cW1, cb1, cW2, cb2,
           eW1, eb1, eW2, eb2, eg, ebt, eps, mW1, mb1, mW2, mb2, mg, mbt,
           l1W, l1b, l2W, l2b, l3W, l3b, l4W, l4b):
    src = edge_index[0]
    dst = edge_index[1]

    def row(v):
        return v.reshape(1, H)

    vfp = jnp.pad(var_node_features, ((0, 0), (0, H - 3)))
    cfp = jnp.pad(con_node_features, ((0, 0), (0, H - 3)))
    vW1p = jnp.pad(vW1, ((0, H - 3), (0, 0)))
    cW1p = jnp.pad(cW1, ((0, H - 3), (0, 0)))

    n = _enc(vfp, vW1p, row(vb1), vW2, row(vb2))
    c = _enc(cfp, cW1p, row(cb1), cW2, row(cb2))
    x = jnp.concatenate([n, c], axis=0)

    ea3 = edge_features.reshape(NEB, ER, 128)
    h2e, abe = _eemb(ea3, eW1, row(eb1), eW2, row(eb2), row(eg), row(ebt))

    epsb = jnp.broadcast_to(eps.reshape(1, 1).astype(_f32), (1, H))

    h2e = h2e.reshape(E * H)
    xs = [x]
    for _ in range(4):
        aggp = _mp_call(x, h2e, src, dst, abe)
        h2n, abn = _upd(x, aggp, epsb, mW1, row(mb1), mW2, row(mb2),
                        row(mg), row(mbt))
        x = _apply(h2n, abn)
        xs.append(x)

    l4Wp = jnp.pad(l4W, ((0, 0), (0, H - 2)))
    l4bp = jnp.pad(l4b, ((0, H - 2),)).reshape(1, H)
    outp = _head(xs[0][:NV], xs[1][:NV], xs[2][:NV], xs[3][:NV], xs[4][:NV],
                 l1W, row(l1b), l2W, row(l2b), l3W, row(l3b), l4Wp, l4bp)
    return outp[:, :2]


# speedup vs baseline: 2.8026x; 2.8026x over previous
import functools

import jax
import jax.numpy as jnp
from jax import lax
from jax.experimental import pallas as pl
from jax.experimental.pallas import tpu as pltpu
from jax.experimental.pallas import tpu_sc as plsc

NV = 5000
NCON = 5000
NN = 10000
E = 320000
H = 128

RB = 1000
EB = 2560
ER = 20
NEB = E // EB

SC_NC = 2
SC_NS = 16
EC = E // SC_NC
EPW = EC // SC_NS
K = 80
NCHUNK = EPW // K
RPS = 624

_f32 = jnp.float32


def _full(shape):
    return pl.BlockSpec(shape, lambda i: tuple(0 for _ in shape))


def _enc_body(x_ref, w1_ref, b1_ref, w2_ref, b2_ref, o_ref):
    h = jnp.maximum(
        jnp.dot(x_ref[...], w1_ref[...], preferred_element_type=_f32)
        + b1_ref[...], 0.0)
    o_ref[...] = (
        jnp.dot(h, w2_ref[...], preferred_element_type=_f32) + b2_ref[...])


def _enc(xp, w1p, b1, w2, b2):
    n = xp.shape[0]
    return pl.pallas_call(
        _enc_body,
        grid=(n // RB,),
        in_specs=[
            pl.BlockSpec((RB, H), lambda i: (i, 0)),
            _full((H, H)), _full((1, H)), _full((H, H)), _full((1, H)),
        ],
        out_specs=pl.BlockSpec((RB, H), lambda i: (i, 0)),
        out_shape=jax.ShapeDtypeStruct((n, H), _f32),
    )(xp, w1p, b1, w2, b2)


def _eemb_body(ea_ref, w1_ref, b1_ref, w2_ref, b2_ref, g_ref, bt_ref,
               h2_ref, ab_ref, acc_ref):
    i = pl.program_id(0)

    @pl.when(i == 0)
    def _():
        acc_ref[...] = jnp.zeros((8, H), _f32)

    a = ea_ref[0]
    abc = jnp.broadcast_to(a[:, :, None], (ER, 128, H)).reshape(EB, H)
    h1 = jnp.maximum(abc * w1_ref[...] + b1_ref[...], 0.0)
    h2 = jnp.maximum(
        jnp.dot(h1, w2_ref[...], preferred_element_type=_f32)
        + b2_ref[...], 0.0)
    h2_ref[...] = h2
    acc_ref[0:1] = acc_ref[0:1] + jnp.sum(h2, axis=0, keepdims=True)
    acc_ref[1:2] = acc_ref[1:2] + jnp.sum(h2 * h2, axis=0, keepdims=True)

    @pl.when(i == NEB - 1)
    def _():
        mu = acc_ref[0:1] / E
        var = acc_ref[1:2] / E - mu * mu
        aa = g_ref[...] * lax.rsqrt(var + 1e-5)
        bb = bt_ref[...] - mu * aa
        ab_ref[...] = jnp.concatenate(
            [aa, bb, jnp.zeros((6, H), _f32)], axis=0)


def _eemb(ea3, w1, b1, w2, b2, g, bt):
    return pl.pallas_call(
        _eemb_body,
        grid=(NEB,),
        in_specs=[
            pl.BlockSpec((1, ER, 128), lambda i: (i, 0, 0)),
            _full((1, H)), _full((1, H)), _full((H, H)), _full((1, H)),
            _full((1, H)), _full((1, H)),
        ],
        out_specs=[
            pl.BlockSpec((EB, H), lambda i: (i, 0)),
            _full((8, H)),
        ],
        out_shape=[
            jax.ShapeDtypeStruct((E, H), _f32),
            jax.ShapeDtypeStruct((8, H), _f32),
        ],
        scratch_shapes=[pltpu.VMEM((8, H), _f32)],
    )(ea3, w1, b1, w2, b2, g, bt)


def _upd_body(x_ref, agg_ref, eps_ref, w1_ref, b1_ref, w2_ref, b2_ref,
              g_ref, bt_ref, h2_ref, ab_ref, acc_ref):
    i = pl.program_id(0)

    @pl.when(i == 0)
    def _():
        acc_ref[...] = jnp.zeros((8, H), _f32)

    pre = x_ref[...] * (1.0 + eps_ref[...]) + agg_ref[0] + agg_ref[1]
    h1 = jnp.maximum(
        jnp.dot(pre, w1_ref[...], preferred_element_type=_f32)
        + b1_ref[...], 0.0)
    h2 = jnp.maximum(
        jnp.dot(h1, w2_ref[...], preferred_element_type=_f32)
        + b2_ref[...], 0.0)
    h2_ref[...] = h2
    acc_ref[0:1] = acc_ref[0:1] + jnp.sum(h2, axis=0, keepdims=True)
    acc_ref[1:2] = acc_ref[1:2] + jnp.sum(h2 * h2, axis=0, keepdims=True)

    @pl.when(i == NN // RB - 1)
    def _():
        mu = acc_ref[0:1] / NN
        var = acc_ref[1:2] / NN - mu * mu
        aa = g_ref[...] * lax.rsqrt(var + 1e-5)
        bb = bt_ref[...] - mu * aa
        ab_ref[...] = jnp.concatenate(
            [aa, bb, jnp.zeros((6, H), _f32)], axis=0)


def _upd(x, aggp, epsb, w1, b1, w2, b2, g, bt):
    return pl.pallas_call(
        _upd_body,
        grid=(NN // RB,),
        in_specs=[
            pl.BlockSpec((RB, H), lambda i: (i, 0)),
            pl.BlockSpec((2, RB, H), lambda i: (0, i, 0)),
            _full((1, H)),
            _full((H, H)), _full((1, H)), _full((H, H)), _full((1, H)),
            _full((1, H)), _full((1, H)),
        ],
        out_specs=[
            pl.BlockSpec((RB, H), lambda i: (i, 0)),
            _full((8, H)),
        ],
        out_shape=[
            jax.ShapeDtypeStruct((NN, H), _f32),
            jax.ShapeDtypeStruct((8, H), _f32),
        ],
        scratch_shapes=[pltpu.VMEM((8, H), _f32)],
    )(x, aggp, epsb, w1, b1, w2, b2, g, bt)


def _apply_body(h_ref, ab_ref, o_ref):
    o_ref[...] = jnp.maximum(h_ref[...] * ab_ref[0:1] + ab_ref[1:2], 0.0)


def _apply(h, ab):
    return pl.pallas_call(
        _apply_body,
        grid=(NN // RB,),
        in_specs=[
            pl.BlockSpec((RB, H), lambda i: (i, 0)),
            _full((8, H)),
        ],
        out_specs=pl.BlockSpec((RB, H), lambda i: (i, 0)),
        out_shape=jax.ShapeDtypeStruct((NN, H), _f32),
    )(h, ab)


def _head_body(x0_ref, x1_ref, x2_ref, x3_ref, x4_ref, w1_ref, b1_ref,
               w2_ref, b2_ref, w3_ref, b3_ref, w4_ref, b4_ref, o_ref):
    w = w1_ref[...]
    h = (jnp.dot(x0_ref[...], w[0:H], preferred_element_type=_f32)
         + jnp.dot(x1_ref[...], w[H:2 * H], preferred_element_type=_f32)
         + jnp.dot(x2_ref[...], w[2 * H:3 * H], preferred_element_type=_f32)
         + jnp.dot(x3_ref[...], w[3 * H:4 * H], preferred_element_type=_f32)
         + jnp.dot(x4_ref[...], w[4 * H:5 * H], preferred_element_type=_f32)
         + b1_ref[...])
    h = jnp.maximum(h, 0.0)
    h = jnp.maximum(
        jnp.dot(h, w2_ref[...], preferred_element_type=_f32)
        + b2_ref[...], 0.0)
    h = jnp.maximum(
        jnp.dot(h, w3_ref[...], preferred_element_type=_f32)
        + b3_ref[...], 0.0)
    of = jnp.dot(h, w4_ref[...], preferred_element_type=_f32)
    o0 = of[:, 0:1] + b4_ref[:, 0:1]
    o1 = of[:, 1:2] + b4_ref[:, 1:2]
    m = jnp.maximum(o0, o1)
    ls = m + jnp.log(jnp.exp(o0 - m) + jnp.exp(o1 - m))
    o_ref[...] = jnp.concatenate(
        [o0 - ls, o1 - ls, jnp.zeros((RB, H - 2), _f32)], axis=1)


def _head(x0, x1, x2, x3, x4, w1, b1, w2, b2, w3, b3, w4p, b4p):
    blk = pl.BlockSpec((RB, H), lambda i: (i, 0))
    return pl.pallas_call(
        _head_body,
        grid=(NV // RB,),
        in_specs=[
            blk, blk, blk, blk, blk,
            _full((5 * H, H)), _full((1, H)),
            _full((H, H)), _full((1, H)),
            _full((H, H)), _full((1, H)),
            _full((H, H)), _full((1, H)),
        ],
        out_specs=pl.BlockSpec((RB, H), lambda i: (i, 0)),
        out_shape=jax.ShapeDtypeStruct((NV, H), _f32),
    )(x0, x1, x2, x3, x4, w1, b1, w2, b2, w3, b3, w4p, b4p)


def _mp_body(x_hbm, h2e_hbm, src_hbm, dst_hbm, ab_hbm, out_hbm,
             sidx_a, sidx_b, didx_v, gx_a, he_a, gx_b, he_b,
             ab_v, agg_sh, s_si_a, s_si_b, s_di, s_g_a, s_g_b, s_h_a, s_h_b):
    cid = lax.axis_index("c")
    sid = lax.axis_index("s")

    def _zrow(r, carry):
        for c in range(H // 16):
            gx_a[r, pl.ds(c * 16, 16)] = jnp.zeros((16,), _f32)
        return carry
    lax.fori_loop(0, K, _zrow, 0)
    start = sid * RPS
    for j in range(RPS // K):
        pltpu.sync_copy(gx_a, agg_sh.at[pl.ds(start + j * K, K)])
    pltpu.sync_copy(gx_a.at[pl.ds(0, RPS - (RPS // K) * K)],
                    agg_sh.at[pl.ds(start + (RPS // K) * K,
                                    RPS - (RPS // K) * K)])

    @pl.when(sid == SC_NS - 1)
    def _():
        pltpu.sync_copy(gx_a.at[pl.ds(0, NN - SC_NS * RPS)],
                        agg_sh.at[pl.ds(SC_NS * RPS, NN - SC_NS * RPS)])

    pltpu.sync_copy(ab_hbm, ab_v)
    avs = [ab_v[0, pl.ds(c * 16, 16)] for c in range(H // 16)]
    bvs = [ab_v[1, pl.ds(c * 16, 16)] for c in range(H // 16)]

    plsc.subcore_barrier()

    base = cid * EC + sid * EPW

    def issue_si(i, si, sem):
        pltpu.async_copy(src_hbm.at[pl.ds(base + i * K, K)], si, sem)

    def wait_si(si, sem):
        pltpu.make_async_copy(src_hbm.at[pl.ds(0, K)], si, sem).wait()

    def issue_di(i):
        pltpu.async_copy(dst_hbm.at[pl.ds(base + i * K, K)], didx_v, s_di)

    def issue_g(i, si, gx, he, sg, sh):
        pltpu.async_copy(x_hbm.at[si], gx, sg)
        pltpu.async_copy(h2e_hbm.at[pl.ds(base + i * K, K)], he, sh)

    def msg_loop(gx, he):
        def _row(r, c2):
            for c in range(H // 16):
                g = gx[r, pl.ds(c * 16, 16)]
                hh = he[r, pl.ds(c * 16, 16)]
                gx[r, pl.ds(c * 16, 16)] = jnp.maximum(
                    g + (hh * avs[c] + bvs[c]), 0.0)
            return c2
        lax.fori_loop(0, K, _row, 0)

    def compute_scatter(gx, he, sg, sh):
        pltpu.make_async_copy(x_hbm.at[pl.ds(0, K)], gx, sg).wait()
        pltpu.make_async_copy(h2e_hbm.at[pl.ds(0, K)], he, sh).wait()
        msg_loop(gx, he)
        pltpu.make_async_copy(dst_hbm.at[pl.ds(0, K)], didx_v, s_di).wait()
        pltpu.sync_copy(gx, agg_sh.at[didx_v], add=True)

    issue_si(0, sidx_a, s_si_a)
    issue_si(1, sidx_b, s_si_b)
    issue_di(0)
    wait_si(sidx_a, s_si_a)
    issue_g(0, sidx_a, gx_a, he_a, s_g_a, s_h_a)

    def _pair(j, carry):
        i0 = 2 * j
        i1 = i0 + 1
        wait_si(sidx_b, s_si_b)
        issue_g(i1, sidx_b, gx_b, he_b, s_g_b, s_h_b)
        pltpu.make_async_copy(x_hbm.at[pl.ds(0, K)], gx_a, s_g_a).wait()
        issue_si(i0 + 2, sidx_a, s_si_a)
        pltpu.make_async_copy(h2e_hbm.at[pl.ds(0, K)], he_a, s_h_a).wait()
        msg_loop(gx_a, he_a)
        pltpu.make_async_copy(dst_hbm.at[pl.ds(0, K)], didx_v, s_di).wait()
        pltpu.sync_copy(gx_a, agg_sh.at[didx_v], add=True)
        issue_di(i1)
        wait_si(sidx_a, s_si_a)
        issue_g(i0 + 2, sidx_a, gx_a, he_a, s_g_a, s_h_a)
        pltpu.make_async_copy(x_hbm.at[pl.ds(0, K)], gx_b, s_g_b).wait()

        @pl.when(i1 + 2 < NCHUNK)
        def _():
            issue_si(i1 + 2, sidx_b, s_si_b)
        pltpu.make_async_copy(h2e_hbm.at[pl.ds(0, K)], he_b, s_h_b).wait()
        msg_loop(gx_b, he_b)
        pltpu.make_async_copy(dst_hbm.at[pl.ds(0, K)], didx_v, s_di).wait()
        pltpu.sync_copy(gx_b, agg_sh.at[didx_v], add=True)
        issue_di(i0 + 2)
        return carry
    lax.fori_loop(0, (NCHUNK - 1) // 2, _pair, 0)

    compute_scatter(gx_a, he_a, s_g_a, s_h_a)

    plsc.subcore_barrier()

    nfull = RPS // K
    for j in range(nfull):
        r0 = start + j * K
        pltpu.sync_copy(agg_sh.at[pl.ds(r0, K)], gx_a)
        pltpu.sync_copy(gx_a, out_hbm.at[cid, pl.ds(r0, K)])
    rpart = RPS - nfull * K
    r0 = start + nfull * K
    pltpu.sync_copy(agg_sh.at[pl.ds(r0, rpart)], gx_a.at[pl.ds(0, rpart)])
    pltpu.sync_copy(gx_a.at[pl.ds(0, rpart)], out_hbm.at[cid, pl.ds(r0, rpart)])

    @pl.when(sid == SC_NS - 1)
    def _():
        rem = NN - SC_NS * RPS
        pltpu.sync_copy(agg_sh.at[pl.ds(SC_NS * RPS, rem)],
                        gx_b.at[pl.ds(0, rem)])
        pltpu.sync_copy(gx_b.at[pl.ds(0, rem)],
                        out_hbm.at[cid, pl.ds(SC_NS * RPS, rem)])


@functools.lru_cache(maxsize=1)
def _mp_build():
    mesh = plsc.VectorSubcoreMesh(
        core_axis_name="c", subcore_axis_name="s",
        num_cores=SC_NC, num_subcores=SC_NS)
    return pl.kernel(
        _mp_body,
        out_type=jax.ShapeDtypeStruct((SC_NC, NN, H), _f32),
        mesh=mesh,
        scratch_types=[
            pltpu.VMEM((K,), jnp.int32),
            pltpu.VMEM((K,), jnp.int32),
            pltpu.VMEM((K,), jnp.int32),
            pltpu.VMEM((K, H), _f32),
            pltpu.VMEM((K, H), _f32),
            pltpu.VMEM((K, H), _f32),
            pltpu.VMEM((K, H), _f32),
            pltpu.VMEM((8, H), _f32),
            pltpu.VMEM_SHARED((NN, H), _f32),
            pltpu.SemaphoreType.DMA,
            pltpu.SemaphoreType.DMA,
            pltpu.SemaphoreType.DMA,
            pltpu.SemaphoreType.DMA,
            pltpu.SemaphoreType.DMA,
            pltpu.SemaphoreType.DMA,
            pltpu.SemaphoreType.DMA,
        ],
    )


def _mp_call(x, h2e, src, dst, abe):
    return _mp_build()(x, h2e, src, dst, abe)


def kernel(var_node_features, con_node_features, edge_features, edge_index,
           assoc_var, assoc_con, vW1, vb1, vW2, vb2, cW1, cb1, cW2, cb2,
           eW1, eb1, eW2, eb2, eg, ebt, eps, mW1, mb1, mW2, mb2, mg, mbt,
           l1W, l1b, l2W, l2b, l3W, l3b, l4W, l4b):
    src = edge_index[0]
    dst = edge_index[1]

    def row(v):
        return v.reshape(1, H)

    vfp = jnp.pad(var_node_features, ((0, 0), (0, H - 3)))
    cfp = jnp.pad(con_node_features, ((0, 0), (0, H - 3)))
    vW1p = jnp.pad(vW1, ((0, H - 3), (0, 0)))
    cW1p = jnp.pad(cW1, ((0, H - 3), (0, 0)))

    n = _enc(vfp, vW1p, row(vb1), vW2, row(vb2))
    c = _enc(cfp, cW1p, row(cb1), cW2, row(cb2))
    x = jnp.concatenate([n, c], axis=0)

    ea3 = edge_features.reshape(NEB, ER, 128)
    h2e, abe = _eemb(ea3, eW1, row(eb1), eW2, row(eb2), row(eg), row(ebt))

    epsb = jnp.broadcast_to(eps.reshape(1, 1).astype(_f32), (1, H))

    xs = [x]
    for _ in range(4):
        aggp = _mp_call(x, h2e, src, dst, abe)
        h2n, abn = _upd(x, aggp, epsb, mW1, row(mb1), mW2, row(mb2),
                        row(mg), row(mbt))
        x = _apply(h2n, abn)
        xs.append(x)

    l4Wp = jnp.pad(l4W, ((0, 0), (0, H - 2)))
    l4bp = jnp.pad(l4b, ((0, H - 2),)).reshape(1, H)
    outp = _head(xs[0][:NV], xs[1][:NV], xs[2][:NV], xs[3][:NV], xs[4][:NV],
                 l1W, row(l1b), l2W, row(l2b), l3W, row(l3b), l4Wp, l4bp)
    return outp[:, :2]
